# Initial kernel scaffold; baseline (speedup 1.0000x reference)
import jax
import jax.numpy as jnp
from jax.experimental import pallas as pl


def kernel(rays_o, rays_d, z_vals, weights):
    raise NotImplementedError("write your pallas kernel here")


# speedup vs baseline: 683.3401x; 683.3401x over previous
import jax
import jax.numpy as jnp
from jax.experimental import pallas as pl
from jax.experimental.pallas import tpu as pltpu

N_RAYS = 65536
NC = 64
NS = 128
NZ = NC + NS
NP = 256
NF = 3 * NZ
BLK = 256
BIG = 3.0e38


def _sampler_body(o_ref, d_ref, z_ref, w_ref, pts_ref, zc_ref, s_ref):
    z = z_ref[...]
    w = w_ref[...]
    f32 = jnp.float32
    lane64 = jax.lax.broadcasted_iota(jnp.int32, (1, NC), 1)

    def roll(arr, shift):
        n = arr.shape[1]
        return pltpu.roll(arr, shift % n, axis=1)

    wp = jnp.where((lane64 >= 1) & (lane64 <= NC - 2), w + 1e-5, 0.0)
    acc = wp
    for s in (1, 2, 4, 8, 16, 32):
        rolled = roll(acc, s)
        acc = acc + jnp.where(lane64 >= s, rolled, 0.0)
    total = acc[:, NC - 1:NC]
    cdf = acc / total

    cdf_next = roll(cdf, -1)
    zn = roll(z, -1)
    bins = 0.5 * (z + zn)
    binsn = roll(bins, -1)
    bins_last = bins[:, NC - 2:NC - 1]
    binsn = jnp.where(lane64 >= NC - 2, bins_last, binsn)
    denom = cdf_next - cdf
    denomg = jnp.where(denom < 1e-5, 1.0, denom)
    B = (binsn - bins) / denomg
    A = bins - cdf * B

    u = jax.lax.broadcasted_iota(jnp.int32, (1, NS), 1).astype(f32) * (
        1.0 / (NS - 1))
    samples = jnp.zeros((BLK, NS), dtype=f32)
    ge = jnp.ones((1, NS), dtype=jnp.bool_)
    for k in range(NC - 1):
        if k < NC - 2:
            ge_next = u >= cdf[:, k + 1:k + 2]
            b = ge & (~ge_next)
        else:
            b = ge
            ge_next = None
        samples = jnp.where(b, A[:, k:k + 1] + B[:, k:k + 1] * u, samples)
        ge = ge_next
    s_ref[...] = samples

    lane256 = jax.lax.broadcasted_iota(jnp.int32, (1, NP), 1)
    x = jnp.concatenate(
        [z, samples, jnp.full((BLK, NP - NZ), BIG, dtype=f32)], axis=1)
    for k in range(1, 9):
        size = 1 << k
        for j in range(k - 1, -1, -1):
            d = 1 << j
            lowmask = (lane256 & d) == 0
            take_min = lowmask == ((lane256 & size) == 0)
            partner = jnp.where(
                lowmask, roll(x, -d), roll(x, d))
            x = jnp.where(take_min, jnp.minimum(x, partner),
                          jnp.maximum(x, partner))
    zc = x[:, :NZ]
    zc_ref[...] = zc

    row_iota = jax.lax.broadcasted_iota(jnp.int32, (NZ, NF), 0)
    col_iota = jax.lax.broadcasted_iota(jnp.int32, (NZ, NF), 1)
    G = (col_iota // 3 == row_iota).astype(f32)
    z3 = jnp.dot(zc, G, preferred_element_type=f32,
                 precision=jax.lax.Precision.HIGHEST)
    lane576 = jax.lax.broadcasted_iota(jnp.int32, (1, NF), 1)
    comp = lane576 % 3
    pts = jnp.zeros((BLK, NF), dtype=f32)
    for c in range(3):
        val = o_ref[:, c:c + 1] + d_ref[:, c:c + 1] * z3
        pts = jnp.where(comp == c, val, pts)
    pts_ref[...] = pts


def _run(rays_o, rays_d, z_vals, weights, interpret=False):
    n = z_vals.shape[0]
    grid = (n // BLK,)
    out_shape = [
        jax.ShapeDtypeStruct((n, NF), jnp.float32),
        jax.ShapeDtypeStruct((n, NZ), jnp.float32),
        jax.ShapeDtypeStruct((n, NS), jnp.float32),
    ]
    pts_flat, zc, samples = pl.pallas_call(
        _sampler_body,
        grid=grid,
        in_specs=[
            pl.BlockSpec((BLK, 3), lambda i: (i, 0)),
            pl.BlockSpec((BLK, 3), lambda i: (i, 0)),
            pl.BlockSpec((BLK, NC), lambda i: (i, 0)),
            pl.BlockSpec((BLK, NC), lambda i: (i, 0)),
        ],
        out_specs=[
            pl.BlockSpec((BLK, NF), lambda i: (i, 0)),
            pl.BlockSpec((BLK, NZ), lambda i: (i, 0)),
            pl.BlockSpec((BLK, NS), lambda i: (i, 0)),
        ],
        out_shape=out_shape,
        compiler_params=pltpu.CompilerParams(
            dimension_semantics=("arbitrary",)),
        interpret=interpret,
    )(rays_o, rays_d, z_vals, weights)
    pts = pts_flat.reshape(n, NZ, 3)
    return pts, zc, samples


def kernel(rays_o, rays_d, z_vals, weights):
    return _run(rays_o, rays_d, z_vals, weights)


# speedup vs baseline: 1333.9281x; 1.9521x over previous
import jax
import jax.numpy as jnp
import numpy as np
from jax.experimental import pallas as pl
from jax.experimental.pallas import tpu as pltpu

N_RAYS = 65536
NC = 64
NS = 128
NZ = NC + NS
NP = 256
NF = 3 * NZ
BLK = 512
BIG = 3.0e38


def _sampler_body(o_ref, d_ref, z_ref, w_ref, g_ref, h_ref, pts_ref, zc_ref, s_ref):
    z = z_ref[...]
    w = w_ref[...]
    f32 = jnp.float32
    lane64 = jax.lax.broadcasted_iota(jnp.int32, (1, NC), 1)

    def roll(arr, shift):
        n = arr.shape[1]
        return pltpu.roll(arr, shift % n, axis=1)

    wp = jnp.where((lane64 >= 1) & (lane64 <= NC - 2), w + 1e-5, 0.0)
    acc = wp
    for s in (1, 2, 4, 8, 16, 32):
        rolled = roll(acc, s)
        acc = acc + jnp.where(lane64 >= s, rolled, 0.0)
    total = acc[:, NC - 1:NC]
    cdf = acc / total

    cdf_next = roll(cdf, -1)
    zn = roll(z, -1)
    bins = 0.5 * (z + zn)
    binsn = roll(bins, -1)
    bins_last = bins[:, NC - 2:NC - 1]
    binsn = jnp.where(lane64 >= NC - 2, bins_last, binsn)
    denom = cdf_next - cdf
    denomg = jnp.where(denom < 1e-5, 1.0, denom)
    B = (binsn - bins) / denomg
    A = bins - cdf * B

    cdf_t = jnp.swapaxes(cdf, 0, 1)
    A_t = jnp.swapaxes(A, 0, 1)
    B_t = jnp.swapaxes(B, 0, 1)
    u_t = jax.lax.broadcasted_iota(jnp.int32, (NS, 1), 0).astype(f32) * (
        1.0 / (NS - 1))
    Ag = jnp.broadcast_to(A_t[0:1, :], (NS, BLK))
    Bg = jnp.broadcast_to(B_t[0:1, :], (NS, BLK))
    for k in range(1, NC - 1):
        ge = u_t >= cdf_t[k:k + 1, :]
        Ag = jnp.where(ge, A_t[k:k + 1, :], Ag)
        Bg = jnp.where(ge, B_t[k:k + 1, :], Bg)
    samples = jnp.swapaxes(Ag + Bg * u_t, 0, 1)
    s_ref[...] = samples

    lane128 = jax.lax.broadcasted_iota(jnp.int32, (1, NS), 1)
    x0 = jnp.concatenate([z, jnp.full((BLK, NS - NC), BIG, dtype=f32)],
                         axis=1)
    x1 = samples
    for k in range(1, 9):
        size = 1 << k
        for j in range(k - 1, -1, -1):
            d = 1 << j
            if d == NS:
                lo = jnp.minimum(x0, x1)
                x1 = jnp.maximum(x0, x1)
                x0 = lo
                continue
            idx = jnp.broadcast_to(lane128 ^ d, (BLK, NS))
            low = (lane128 & d) == 0
            tm0 = low == ((lane128 & size) == 0)
            tm1 = low == (((lane128 + NS) & size) == 0)
            p0 = jnp.take_along_axis(x0, idx, axis=1)
            p1 = jnp.take_along_axis(x1, idx, axis=1)
            x0 = jnp.where(tm0, jnp.minimum(x0, p0), jnp.maximum(x0, p0))
            x1 = jnp.where(tm1, jnp.minimum(x1, p1), jnp.maximum(x1, p1))
    zc = jnp.concatenate([x0, x1[:, :NZ - NS]], axis=1)
    zc_ref[...] = zc

    hi = jax.lax.Precision.HIGHEST
    z3 = jnp.dot(zc, g_ref[...], preferred_element_type=f32, precision=hi)
    o3 = jnp.dot(o_ref[...], h_ref[...], preferred_element_type=f32,
                 precision=hi)
    d3 = jnp.dot(d_ref[...], h_ref[...], preferred_element_type=f32,
                 precision=hi)
    pts_ref[...] = o3 + d3 * z3


def _run(rays_o, rays_d, z_vals, weights, interpret=False):
    n = z_vals.shape[0]
    grid = (n // BLK,)
    s_iota = np.arange(NZ)[:, None]
    t_iota = np.arange(NF)[None, :]
    g_mat = jnp.asarray(t_iota // 3 == s_iota, dtype=jnp.float32)
    h_mat = jnp.asarray(t_iota % 3 == np.arange(3)[:, None],
                        dtype=jnp.float32)
    out_shape = [
        jax.ShapeDtypeStruct((n, NF), jnp.float32),
        jax.ShapeDtypeStruct((n, NZ), jnp.float32),
        jax.ShapeDtypeStruct((n, NS), jnp.float32),
    ]
    pts_flat, zc, samples = pl.pallas_call(
        _sampler_body,
        grid=grid,
        in_specs=[
            pl.BlockSpec((BLK, 3), lambda i: (i, 0)),
            pl.BlockSpec((BLK, 3), lambda i: (i, 0)),
            pl.BlockSpec((BLK, NC), lambda i: (i, 0)),
            pl.BlockSpec((BLK, NC), lambda i: (i, 0)),
            pl.BlockSpec((NZ, NF), lambda i: (0, 0)),
            pl.BlockSpec((3, NF), lambda i: (0, 0)),
        ],
        out_specs=[
            pl.BlockSpec((BLK, NF), lambda i: (i, 0)),
            pl.BlockSpec((BLK, NZ), lambda i: (i, 0)),
            pl.BlockSpec((BLK, NS), lambda i: (i, 0)),
        ],
        out_shape=out_shape,
        compiler_params=pltpu.CompilerParams(
            dimension_semantics=("arbitrary",)),
        interpret=interpret,
    )(rays_o, rays_d, z_vals, weights, g_mat, h_mat)
    pts = pts_flat.reshape(n, NZ, 3)
    return pts, zc, samples


def kernel(rays_o, rays_d, z_vals, weights):
    return _run(rays_o, rays_d, z_vals, weights)


# speedup vs baseline: 1962.6984x; 1.4714x over previous
import jax
import jax.numpy as jnp
import numpy as np
from jax.experimental import pallas as pl
from jax.experimental.pallas import tpu as pltpu

N_RAYS = 65536
NC = 64
NS = 128
NZ = NC + NS
NF = 3 * NZ
BLK = 512
SUB = 256
BIG = 3.0e38


def _sampler_body(o_ref, d_ref, z_ref, zt_ref, wt_ref, cm_ref, bm_ref,
                  g_ref, h_ref, pts_ref, zc_ref, s_ref):
    f32 = jnp.float32
    hi = jax.lax.Precision.HIGHEST
    z = z_ref[...]
    z_t = zt_ref[...]
    w_t = wt_ref[...]

    sub64 = jax.lax.broadcasted_iota(jnp.int32, (NC, 1), 0)
    wp_t = jnp.where((sub64 >= 1) & (sub64 <= NC - 2), w_t + 1e-5, 0.0)
    cums = jnp.dot(cm_ref[...], wp_t, preferred_element_type=f32,
                   precision=hi)
    rtot = 1.0 / cums[NC - 1:NC, :]
    cdf_t = cums[:NC] * rtot
    cdfn_t = cums[NC:] * rtot
    binsb = jnp.dot(bm_ref[...], z_t, preferred_element_type=f32,
                    precision=hi)
    bins_t = binsb[:NC]
    binsn_t = binsb[NC:]
    denom = cdfn_t - cdf_t
    denomg = jnp.where(denom < 1e-5, 1.0, denom)
    B_t = (binsn_t - bins_t) / denomg
    A_t = bins_t - cdf_t * B_t

    u_t = jax.lax.broadcasted_iota(jnp.int32, (NS, 1), 0).astype(f32) * (
        1.0 / (NS - 1))
    bounds = (0, 16, 32, 48, NC - 1)
    ags, bgs = [], []
    for c in range(4):
        k0 = bounds[c]
        Ac = jnp.broadcast_to(A_t[k0:k0 + 1, :], (NS, BLK))
        Bc = jnp.broadcast_to(B_t[k0:k0 + 1, :], (NS, BLK))
        for k in range(k0 + 1, bounds[c + 1]):
            ge = u_t >= cdf_t[k:k + 1, :]
            Ac = jnp.where(ge, A_t[k:k + 1, :], Ac)
            Bc = jnp.where(ge, B_t[k:k + 1, :], Bc)
        ags.append(Ac)
        bgs.append(Bc)
    ge16 = u_t >= cdf_t[16:17, :]
    ge32 = u_t >= cdf_t[32:33, :]
    ge48 = u_t >= cdf_t[48:49, :]
    Ag = jnp.where(ge32, jnp.where(ge48, ags[3], ags[2]),
                   jnp.where(ge16, ags[1], ags[0]))
    Bg = jnp.where(ge32, jnp.where(ge48, bgs[3], bgs[2]),
                   jnp.where(ge16, bgs[1], bgs[0]))
    samples = jnp.swapaxes(Ag + Bg * u_t, 0, 1)
    s_ref[...] = samples

    lane128 = jax.lax.broadcasted_iota(jnp.int32, (1, NS), 1)
    xa = jnp.concatenate([z, jnp.full((BLK, NS - NC), BIG, dtype=f32)],
                         axis=1)
    for s0 in range(0, BLK, SUB):
        x0 = xa[s0:s0 + SUB, :]
        x1 = samples[s0:s0 + SUB, :]
        for k in range(1, 9):
            size = 1 << k
            for j in range(k - 1, -1, -1):
                d = 1 << j
                if d == NS:
                    lo = jnp.minimum(x0, x1)
                    x1 = jnp.maximum(x0, x1)
                    x0 = lo
                    continue
                idx = jnp.broadcast_to(lane128 ^ d, (SUB, NS))
                low = (lane128 & d) == 0
                tm0 = low == ((lane128 & size) == 0)
                tm1 = low == (((lane128 + NS) & size) == 0)
                p0 = jnp.take_along_axis(x0, idx, axis=1)
                p1 = jnp.take_along_axis(x1, idx, axis=1)
                x0 = jnp.where(tm0, jnp.minimum(x0, p0), jnp.maximum(x0, p0))
                x1 = jnp.where(tm1, jnp.minimum(x1, p1), jnp.maximum(x1, p1))
        zc = jnp.concatenate([x0, x1[:, :NZ - NS]], axis=1)
        zc_ref[s0:s0 + SUB, :] = zc
        z3 = jnp.dot(zc, g_ref[...], preferred_element_type=f32,
                     precision=hi)
        o3 = jnp.dot(o_ref[s0:s0 + SUB, :], h_ref[...],
                     preferred_element_type=f32, precision=hi)
        d3 = jnp.dot(d_ref[s0:s0 + SUB, :], h_ref[...],
                     preferred_element_type=f32, precision=hi)
        pts_ref[s0:s0 + SUB, :] = o3 + d3 * z3


def _constants():
    j = np.arange(NC)[:, None]
    k = np.arange(NC)[None, :]
    cum = (k <= j).astype(np.float32)
    cumn = (k <= j + 1).astype(np.float32)
    sh = (k == j + 1).astype(np.float32)
    bm = 0.5 * (np.eye(NC, dtype=np.float32) + sh)
    sh2 = sh.copy()
    sh2[NC - 2, :] = 0.0
    sh2[NC - 2, NC - 2] = 1.0
    bn = sh2 @ bm
    cm_mat = np.concatenate([cum, cumn], axis=0)
    bm_mat = np.concatenate([bm, bn], axis=0)
    s_iota = np.arange(NZ)[:, None]
    t_iota = np.arange(NF)[None, :]
    g_mat = (t_iota // 3 == s_iota).astype(np.float32)
    h_mat = (t_iota % 3 == np.arange(3)[:, None]).astype(np.float32)
    return (jnp.asarray(cm_mat), jnp.asarray(bm_mat),
            jnp.asarray(g_mat), jnp.asarray(h_mat))


def _run(rays_o, rays_d, z_vals, weights, interpret=False):
    n = z_vals.shape[0]
    grid = (n // BLK,)
    cm_mat, bm_mat, g_mat, h_mat = _constants()
    z_t = z_vals.T
    w_t = weights.T
    out_shape = [
        jax.ShapeDtypeStruct((n, NF), jnp.float32),
        jax.ShapeDtypeStruct((n, NZ), jnp.float32),
        jax.ShapeDtypeStruct((n, NS), jnp.float32),
    ]
    pts_flat, zc, samples = pl.pallas_call(
        _sampler_body,
        grid=grid,
        in_specs=[
            pl.BlockSpec((BLK, 3), lambda i: (i, 0)),
            pl.BlockSpec((BLK, 3), lambda i: (i, 0)),
            pl.BlockSpec((BLK, NC), lambda i: (i, 0)),
            pl.BlockSpec((NC, BLK), lambda i: (0, i)),
            pl.BlockSpec((NC, BLK), lambda i: (0, i)),
            pl.BlockSpec((2 * NC, NC), lambda i: (0, 0)),
            pl.BlockSpec((2 * NC, NC), lambda i: (0, 0)),
            pl.BlockSpec((NZ, NF), lambda i: (0, 0)),
            pl.BlockSpec((3, NF), lambda i: (0, 0)),
        ],
        out_specs=[
            pl.BlockSpec((BLK, NF), lambda i: (i, 0)),
            pl.BlockSpec((BLK, NZ), lambda i: (i, 0)),
            pl.BlockSpec((BLK, NS), lambda i: (i, 0)),
        ],
        out_shape=out_shape,
        compiler_params=pltpu.CompilerParams(
            dimension_semantics=("arbitrary",)),
        interpret=interpret,
    )(rays_o, rays_d, z_vals, z_t, w_t, cm_mat, bm_mat, g_mat, h_mat)
    pts = pts_flat.reshape(n, NZ, 3)
    return pts, zc, samples


def kernel(rays_o, rays_d, z_vals, weights):
    return _run(rays_o, rays_d, z_vals, weights)


# speedup vs baseline: 1986.1006x; 1.0119x over previous
import jax
import jax.numpy as jnp
import numpy as np
from jax.experimental import pallas as pl
from jax.experimental.pallas import tpu as pltpu

N_RAYS = 65536
NC = 64
NS = 128
NZ = NC + NS
NF = 3 * NZ
BLK = 512
SUB = 256
BIG = 3.0e38


def _sampler_body(o_ref, d_ref, z_ref, zt_ref, wt_ref, cm_ref, bm_ref,
                  g_ref, h_ref, pts_ref, zc_ref, s_ref):
    f32 = jnp.float32
    hi = jax.lax.Precision.HIGHEST
    z = z_ref[...]
    z_t = zt_ref[...]
    w_t = wt_ref[...]

    sub64 = jax.lax.broadcasted_iota(jnp.int32, (NC, 1), 0)
    wp_t = jnp.where((sub64 >= 1) & (sub64 <= NC - 2), w_t + 1e-5, 0.0)
    cums = jnp.dot(cm_ref[...], wp_t, preferred_element_type=f32,
                   precision=hi)
    rtot = 1.0 / cums[NC - 1:NC, :]
    cdf_t = cums[:NC] * rtot
    cdfn_t = cums[NC:] * rtot
    binsb = jnp.dot(bm_ref[...], z_t, preferred_element_type=f32,
                    precision=hi)
    bins_t = binsb[:NC]
    binsn_t = binsb[NC:]
    denom = cdfn_t - cdf_t
    denomg = jnp.where(denom < 1e-5, 1.0, denom)
    B_t = (binsn_t - bins_t) / denomg
    A_t = bins_t - cdf_t * B_t

    u_t = jax.lax.broadcasted_iota(jnp.int32, (NS, 1), 0).astype(f32) * (
        1.0 / (NS - 1))
    bounds = (0, 16, 32, 48, NC - 1)
    ags, bgs = [], []
    for c in range(4):
        k0 = bounds[c]
        Ac = jnp.broadcast_to(A_t[k0:k0 + 1, :], (NS, BLK))
        Bc = jnp.broadcast_to(B_t[k0:k0 + 1, :], (NS, BLK))
        for k in range(k0 + 1, bounds[c + 1]):
            ge = u_t >= cdf_t[k:k + 1, :]
            Ac = jnp.where(ge, A_t[k:k + 1, :], Ac)
            Bc = jnp.where(ge, B_t[k:k + 1, :], Bc)
        ags.append(Ac)
        bgs.append(Bc)
    ge16 = u_t >= cdf_t[16:17, :]
    ge32 = u_t >= cdf_t[32:33, :]
    ge48 = u_t >= cdf_t[48:49, :]
    Ag = jnp.where(ge32, jnp.where(ge48, ags[3], ags[2]),
                   jnp.where(ge16, ags[1], ags[0]))
    Bg = jnp.where(ge32, jnp.where(ge48, bgs[3], bgs[2]),
                   jnp.where(ge16, bgs[1], bgs[0]))
    samples = jnp.swapaxes(Ag + Bg * u_t, 0, 1)
    s_ref[...] = samples

    lane128 = jax.lax.broadcasted_iota(jnp.int32, (1, NS), 1)
    xa = jnp.concatenate([z, jnp.full((BLK, NS - NC), BIG, dtype=f32)],
                         axis=1)
    for s0 in range(0, BLK, SUB):
        x0 = xa[s0:s0 + SUB, :]
        x1 = samples[s0:s0 + SUB, :]
        for k in range(1, 9):
            size = 1 << k
            for j in range(k - 1, -1, -1):
                d = 1 << j
                if d == NS:
                    lo = jnp.minimum(x0, x1)
                    x1 = jnp.maximum(x0, x1)
                    x0 = lo
                    continue
                if d == NS // 2:
                    p0 = pltpu.roll(x0, d, axis=1)
                    p1 = pltpu.roll(x1, d, axis=1)
                    tm0 = ((lane128 & d) == 0) == ((lane128 & size) == 0)
                    tm1 = (((lane128 & d) == 0)
                           == (((lane128 + NS) & size) == 0))
                    x0 = jnp.where(tm0, jnp.minimum(x0, p0),
                                   jnp.maximum(x0, p0))
                    x1 = jnp.where(tm1, jnp.minimum(x1, p1),
                                   jnp.maximum(x1, p1))
                    continue
                idx = jnp.broadcast_to(lane128 ^ d, (SUB, NS))
                low = (lane128 & d) == 0
                tm0 = low == ((lane128 & size) == 0)
                tm1 = low == (((lane128 + NS) & size) == 0)
                p0 = jnp.take_along_axis(x0, idx, axis=1)
                p1 = jnp.take_along_axis(x1, idx, axis=1)
                x0 = jnp.where(tm0, jnp.minimum(x0, p0), jnp.maximum(x0, p0))
                x1 = jnp.where(tm1, jnp.minimum(x1, p1), jnp.maximum(x1, p1))
        zc = jnp.concatenate([x0, x1[:, :NZ - NS]], axis=1)
        zc_ref[s0:s0 + SUB, :] = zc
        def dot2(a, m):
            a_hi = a.astype(jnp.bfloat16)
            a_lo = (a - a_hi.astype(f32)).astype(jnp.bfloat16)
            return (jnp.dot(a_hi, m, preferred_element_type=f32)
                    + jnp.dot(a_lo, m, preferred_element_type=f32))

        gb = g_ref[...].astype(jnp.bfloat16)
        hb = h_ref[...].astype(jnp.bfloat16)
        z3 = dot2(zc, gb)
        o3 = dot2(o_ref[s0:s0 + SUB, :], hb)
        d3 = dot2(d_ref[s0:s0 + SUB, :], hb)
        pts_ref[s0:s0 + SUB, :] = o3 + d3 * z3


def _constants():
    j = np.arange(NC)[:, None]
    k = np.arange(NC)[None, :]
    cum = (k <= j).astype(np.float32)
    cumn = (k <= j + 1).astype(np.float32)
    sh = (k == j + 1).astype(np.float32)
    bm = 0.5 * (np.eye(NC, dtype=np.float32) + sh)
    sh2 = sh.copy()
    sh2[NC - 2, :] = 0.0
    sh2[NC - 2, NC - 2] = 1.0
    bn = sh2 @ bm
    cm_mat = np.concatenate([cum, cumn], axis=0)
    bm_mat = np.concatenate([bm, bn], axis=0)
    s_iota = np.arange(NZ)[:, None]
    t_iota = np.arange(NF)[None, :]
    g_mat = (t_iota // 3 == s_iota).astype(np.float32)
    h_mat = (t_iota % 3 == np.arange(3)[:, None]).astype(np.float32)
    return (jnp.asarray(cm_mat), jnp.asarray(bm_mat),
            jnp.asarray(g_mat), jnp.asarray(h_mat))


def _run(rays_o, rays_d, z_vals, weights, interpret=False):
    n = z_vals.shape[0]
    grid = (n // BLK,)
    cm_mat, bm_mat, g_mat, h_mat = _constants()
    z_t = z_vals.T
    w_t = weights.T
    out_shape = [
        jax.ShapeDtypeStruct((n, NF), jnp.float32),
        jax.ShapeDtypeStruct((n, NZ), jnp.float32),
        jax.ShapeDtypeStruct((n, NS), jnp.float32),
    ]
    pts_flat, zc, samples = pl.pallas_call(
        _sampler_body,
        grid=grid,
        in_specs=[
            pl.BlockSpec((BLK, 3), lambda i: (i, 0)),
            pl.BlockSpec((BLK, 3), lambda i: (i, 0)),
            pl.BlockSpec((BLK, NC), lambda i: (i, 0)),
            pl.BlockSpec((NC, BLK), lambda i: (0, i)),
            pl.BlockSpec((NC, BLK), lambda i: (0, i)),
            pl.BlockSpec((2 * NC, NC), lambda i: (0, 0)),
            pl.BlockSpec((2 * NC, NC), lambda i: (0, 0)),
            pl.BlockSpec((NZ, NF), lambda i: (0, 0)),
            pl.BlockSpec((3, NF), lambda i: (0, 0)),
        ],
        out_specs=[
            pl.BlockSpec((BLK, NF), lambda i: (i, 0)),
            pl.BlockSpec((BLK, NZ), lambda i: (i, 0)),
            pl.BlockSpec((BLK, NS), lambda i: (i, 0)),
        ],
        out_shape=out_shape,
        compiler_params=pltpu.CompilerParams(
            dimension_semantics=("arbitrary",)),
        interpret=interpret,
    )(rays_o, rays_d, z_vals, z_t, w_t, cm_mat, bm_mat, g_mat, h_mat)
    pts = pts_flat.reshape(n, NZ, 3)
    return pts, zc, samples


def kernel(rays_o, rays_d, z_vals, weights):
    return _run(rays_o, rays_d, z_vals, weights)
